# baseline (speedup 1.0000x reference)
import functools

import jax
import jax.numpy as jnp
from jax import lax
from jax.experimental import pallas as pl
from jax.experimental.pallas import tpu as pltpu
from jax.experimental.pallas import tpu_sc as plsc

VEC = 64
VECP = 128
BATCH = 4096
HIST = 200
VOCAB1 = 1000001
CB = 512
TGRID = (VOCAB1 + CB - 1) // CB
VROWS = TGRID * CB
NC, NS = 2, 16
NW = NC * NS
BPW = BATCH // NW
H0 = 128
H1 = HIST - H0


def _tct_body(wt_ref, out_ref):
    t = wt_ref[...].T
    out_ref[:, :VEC] = t
    out_ref[:, VEC:] = t


_tct = pl.pallas_call(
    _tct_body,
    grid=(TGRID,),
    in_specs=[pl.BlockSpec((VEC, CB), lambda j: (0, j))],
    out_specs=pl.BlockSpec((CB, VECP), lambda j: (j, 0)),
    out_shape=jax.ShapeDtypeStruct((VROWS, VECP), jnp.float32),
)


def _emb_body(inds_hbm, w_hbm, out_hbm,
              idx_v, rows0, rows1, gsem0, gsem1, wsem0, wsem1):
    wid = lax.axis_index("s") * NC + lax.axis_index("c")
    b0 = wid * BPW

    rows = (rows0, rows1)
    gsem = (gsem0, gsem1)
    wsem = (wsem0, wsem1)

    pltpu.sync_copy(inds_hbm.at[pl.ds(b0, BPW)], idx_v)

    def fire_gathers(bl, p):
        pltpu.async_copy(
            w_hbm.at[idx_v.at[bl, pl.ds(0, H0)]],
            rows[p].at[pl.ds(0, H0)], gsem[p])
        pltpu.async_copy(
            w_hbm.at[idx_v.at[bl, pl.ds(H0, H1)]],
            rows[p].at[pl.ds(H0, H1)], gsem[p])

    def wait_gathers(p):
        pltpu.make_async_copy(w_hbm.at[pl.ds(0, HIST)], rows[p], gsem[p]).wait()

    def writeback(bl, p):
        pltpu.async_copy(
            rows[p], out_hbm.at[pl.ds((b0 + bl) * HIST, HIST)], wsem[p])

    def wait_writeback(bl, p):
        pltpu.make_async_copy(
            rows[p], out_hbm.at[pl.ds((b0 + bl) * HIST, HIST)], wsem[p]).wait()

    fire_gathers(0, 0)

    @pl.loop(0, BPW, step=2)
    def _steps(g):
        for p in (0, 1):
            bl = g + p
            np_ = 1 - p

            @pl.when(bl + 1 < BPW)
            def _():
                @pl.when(bl >= 1)
                def _():
                    wait_writeback(bl - 1, np_)
                fire_gathers(bl + 1, np_)

            wait_gathers(p)
            writeback(bl, p)

    wait_writeback(BPW - 2, 0)
    wait_writeback(BPW - 1, 1)


_emb = functools.partial(
    pl.kernel,
    out_type=jax.ShapeDtypeStruct((BATCH * HIST, VECP), jnp.float32),
    mesh=plsc.VectorSubcoreMesh(core_axis_name="c", subcore_axis_name="s"),
    scratch_types=[
        pltpu.VMEM((BPW, HIST), jnp.int32),
        pltpu.VMEM((HIST, VECP), jnp.float32),
        pltpu.VMEM((HIST, VECP), jnp.float32),
        pltpu.SemaphoreType.DMA,
        pltpu.SemaphoreType.DMA,
        pltpu.SemaphoreType.DMA,
        pltpu.SemaphoreType.DMA,
    ],
    compiler_params=pltpu.CompilerParams(use_tc_tiling_on_sc=True),
)(_emb_body)


@jax.jit
def kernel(inds, W):
    Wp = _tct(W.T)
    out = _emb(inds, Wp)
    return out[:, :VEC].reshape(BATCH, HIST, VEC)


# speedup vs baseline: 1.7195x; 1.7195x over previous
import functools

import jax
import jax.numpy as jnp
from jax import lax
from jax.experimental import pallas as pl
from jax.experimental.pallas import tpu as pltpu
from jax.experimental.pallas import tpu_sc as plsc

VEC = 64
VECP = 128
BATCH = 4096
HIST = 200
VOCAB1 = 1000001
VROWS = 1000064
NC, NS = 2, 16
NW = NC * NS
BPW = BATCH // NW
H0 = 128
H1 = HIST - H0


def _emb_body(inds_hbm, w_hbm, out_hbm,
              idx_v, rows0, rows1, gsem0, gsem1, wsem0, wsem1):
    wid = lax.axis_index("s") * NC + lax.axis_index("c")
    b0 = wid * BPW

    rows = (rows0, rows1)
    gsem = (gsem0, gsem1)
    wsem = (wsem0, wsem1)

    pltpu.sync_copy(inds_hbm.at[pl.ds(b0, BPW)], idx_v)

    def fire_gathers(bl, p):
        pltpu.async_copy(
            w_hbm.at[idx_v.at[bl, pl.ds(0, H0)]],
            rows[p].at[pl.ds(0, H0)], gsem[p])
        pltpu.async_copy(
            w_hbm.at[idx_v.at[bl, pl.ds(H0, H1)]],
            rows[p].at[pl.ds(H0, H1)], gsem[p])

    def wait_gathers(p):
        pltpu.make_async_copy(w_hbm.at[pl.ds(0, HIST)], rows[p], gsem[p]).wait()

    def writeback(bl, p):
        pltpu.async_copy(
            rows[p], out_hbm.at[pl.ds((b0 + bl) * HIST, HIST)], wsem[p])

    def wait_writeback(bl, p):
        pltpu.make_async_copy(
            rows[p], out_hbm.at[pl.ds((b0 + bl) * HIST, HIST)], wsem[p]).wait()

    fire_gathers(0, 0)

    @pl.loop(0, BPW, step=2)
    def _steps(g):
        for p in (0, 1):
            bl = g + p
            np_ = 1 - p

            @pl.when(bl + 1 < BPW)
            def _():
                @pl.when(bl >= 1)
                def _():
                    wait_writeback(bl - 1, np_)
                fire_gathers(bl + 1, np_)

            wait_gathers(p)
            writeback(bl, p)

    wait_writeback(BPW - 2, 0)
    wait_writeback(BPW - 1, 1)


_emb = functools.partial(
    pl.kernel,
    out_type=jax.ShapeDtypeStruct((BATCH * HIST, VECP), jnp.float32),
    mesh=plsc.VectorSubcoreMesh(core_axis_name="c", subcore_axis_name="s"),
    scratch_types=[
        pltpu.VMEM((BPW, HIST), jnp.int32),
        pltpu.VMEM((HIST, VECP), jnp.float32),
        pltpu.VMEM((HIST, VECP), jnp.float32),
        pltpu.SemaphoreType.DMA,
        pltpu.SemaphoreType.DMA,
        pltpu.SemaphoreType.DMA,
        pltpu.SemaphoreType.DMA,
    ],
    compiler_params=pltpu.CompilerParams(use_tc_tiling_on_sc=True),
)(_emb_body)


@jax.jit
def kernel(inds, W):
    Wp = jnp.pad(W, ((0, VROWS - VOCAB1), (0, VECP - VEC)))
    out = _emb(inds, Wp)
    return out[:, :VEC].reshape(BATCH, HIST, VEC)
